# baseline (speedup 1.0000x reference)
import functools

import jax
import jax.numpy as jnp
from jax import lax
from jax.experimental import pallas as pl
from jax.experimental.pallas import tpu as pltpu
from jax.experimental.pallas import tpu_sc as plsc

N = 10000
E = 320000
D = 128

NC = 2
NS = 16
NW = NC * NS
C = 80
NCH = 126
EPAD = NW * NCH * C
NP = 10240
RPT = NP // NS
BT = 1024

_mesh = plsc.VectorSubcoreMesh(core_axis_name="c", subcore_axis_name="s")


def _make_sc(with_counts: bool):
  out_type = [jax.ShapeDtypeStruct((NC, NP, D), jnp.float32)]
  scratch = [
      pltpu.VMEM_SHARED((NP, D), jnp.float32),
      pltpu.VMEM((NCH, C), jnp.int32),
      pltpu.VMEM((NCH, C), jnp.int32),
      pltpu.VMEM((C, D), jnp.float32),
      pltpu.VMEM((C, D), jnp.float32),
      pltpu.SemaphoreType.DMA,
      pltpu.SemaphoreType.DMA,
      pltpu.SemaphoreType.DMA,
      pltpu.SemaphoreType.DMA,
  ]
  if with_counts:
    out_type.append(jax.ShapeDtypeStruct((NC * NP,), jnp.float32))
    scratch += [
        pltpu.VMEM_SHARED((NP,), jnp.float32),
        pltpu.VMEM((C,), jnp.float32),
        pltpu.VMEM((RPT,), jnp.float32),
        pltpu.SemaphoreType.DMA,
        pltpu.SemaphoreType.DMA,
    ]

  def body(table, src_i, dst_i, acc_out, cnt_out, acc_sh, src_v, dst_v,
           rows0, rows1, sem_g0, sem_g1, sem_s0, sem_s1,
           cnt_sh=None, ones_v=None, cnt_v=None, sem_c0=None, sem_c1=None):
    c = lax.axis_index("c")
    s = lax.axis_index("s")
    w = c * NS + s
    zero16 = jnp.zeros((16,), jnp.float32)

    @pl.loop(0, C)
    def _(i):
      for k in range(D // 16):
        rows0[i, pl.ds(16 * k, 16)] = zero16

    @pl.loop(0, RPT // C)
    def _(k):
      pltpu.sync_copy(rows0, acc_sh.at[pl.ds(s * RPT + k * C, C)])

    pltpu.sync_copy(src_i.at[w], src_v)
    pltpu.sync_copy(dst_i.at[w], dst_v)
    if with_counts:
      @pl.loop(0, RPT // 16)
      def _(i):
        cnt_v[pl.ds(16 * i, 16)] = zero16
      pltpu.sync_copy(cnt_v, cnt_sh.at[pl.ds(s * RPT, RPT)])
      one16 = jnp.ones((16,), jnp.float32)
      for k in range(C // 16):
        ones_v[pl.ds(16 * k, 16)] = one16
    plsc.subcore_barrier()

    rows = (rows0, rows1)
    sem_g = (sem_g0, sem_g1)
    sem_s = (sem_s0, sem_s1)
    sem_c = (sem_c0, sem_c1)

    def wait_g(b, j):
      pltpu.make_async_copy(table.at[src_v.at[j]], rows[b], sem_g[b]).wait()

    def wait_s(b, j):
      pltpu.make_async_copy(
          rows[b], acc_sh.at[dst_v.at[j]], sem_s[b]).wait()

    def wait_c(b, j):
      pltpu.make_async_copy(
          ones_v, cnt_sh.at[dst_v.at[j]], sem_c[b]).wait()

    def issue_sc(b, j):
      pltpu.async_copy(rows[b], acc_sh.at[dst_v.at[j]], sem_s[b], add=True)
      if with_counts:
        pltpu.async_copy(ones_v, cnt_sh.at[dst_v.at[j]], sem_c[b], add=True)

    pltpu.async_copy(table.at[src_v.at[0]], rows0, sem_g0)
    wait_g(0, 0)
    pltpu.async_copy(table.at[src_v.at[1]], rows1, sem_g1)
    issue_sc(0, 0)

    @pl.loop(1, NCH - 1, step=2)
    def _(j):
      wait_g(1, j)
      wait_s(0, j - 1)
      if with_counts:
        wait_c(0, j - 1)
      pltpu.async_copy(table.at[src_v.at[j + 1]], rows0, sem_g0)
      issue_sc(1, j)
      wait_g(0, j + 1)
      wait_s(1, j)
      if with_counts:
        wait_c(1, j)
      pltpu.async_copy(table.at[src_v.at[j + 2]], rows1, sem_g1)
      issue_sc(0, j + 1)

    wait_g(1, NCH - 1)
    wait_s(0, NCH - 2)
    issue_sc(1, NCH - 1)
    wait_s(1, NCH - 1)
    if with_counts:
      wait_c(0, NCH - 2)
      wait_c(1, NCH - 1)
    plsc.subcore_barrier()

    @pl.loop(0, RPT // C)
    def _(k):
      off = s * RPT + k * C
      pltpu.sync_copy(acc_sh.at[pl.ds(off, C)], rows0)
      pltpu.sync_copy(rows0, acc_out.at[c].at[pl.ds(off, C)])
    if with_counts:
      pltpu.sync_copy(cnt_sh.at[pl.ds(s * RPT, RPT)], cnt_v)
      pltpu.sync_copy(cnt_v, cnt_out.at[pl.ds(c * NP + s * RPT, RPT)])

  if with_counts:
    def body_wc(table, src_i, dst_i, acc_out, cnt_out, acc_sh, src_v,
                dst_v, rows0, rows1, sem_g0, sem_g1, sem_s0, sem_s1,
                cnt_sh, ones_v, cnt_v, sem_c0, sem_c1):
      body(table, src_i, dst_i, acc_out, cnt_out, acc_sh, src_v, dst_v,
           rows0, rows1, sem_g0, sem_g1, sem_s0, sem_s1,
           cnt_sh, ones_v, cnt_v, sem_c0, sem_c1)
    fn = body_wc
  else:
    def body_nc(table, src_i, dst_i, acc_out, acc_sh, src_v, dst_v,
                rows0, rows1, sem_g0, sem_g1, sem_s0, sem_s1):
      body(table, src_i, dst_i, acc_out, None, acc_sh, src_v, dst_v,
           rows0, rows1, sem_g0, sem_g1, sem_s0, sem_s1)
    fn = body_nc

  return pl.kernel(
      fn, out_type=out_type, mesh=_mesh, scratch_types=scratch,
      compiler_params=pltpu.CompilerParams(use_tc_tiling_on_sc=False),
      name="sc_agg_cnt" if with_counts else "sc_agg")


_sc_agg_counts = _make_sc(True)
_sc_agg = _make_sc(False)


def _tc_combine_body(acc_ref, cnt_ref, h_ref, wl_ref, wr_ref, b_ref, out_ref):
  agg = acc_ref[0] + acc_ref[1]
  cnt = jnp.sum(cnt_ref[...], axis=0)[:, None]
  mean = agg * (1.0 / jnp.maximum(cnt, 1.0))
  dn = (((1,), (1,)), ((), ()))
  out_ref[...] = (
      lax.dot_general(mean, wl_ref[...], dn, preferred_element_type=jnp.float32)
      + lax.dot_general(h_ref[...], wr_ref[...], dn,
                        preferred_element_type=jnp.float32)
      + b_ref[...])


_tc_combine = pl.pallas_call(
    _tc_combine_body,
    grid=(NP // BT,),
    in_specs=[
        pl.BlockSpec((NC, BT, D), lambda i: (0, i, 0)),
        pl.BlockSpec((NC, BT), lambda i: (0, i)),
        pl.BlockSpec((BT, D), lambda i: (i, 0)),
        pl.BlockSpec((D, D), lambda i: (0, 0)),
        pl.BlockSpec((D, D), lambda i: (0, 0)),
        pl.BlockSpec((1, D), lambda i: (0, 0)),
    ],
    out_specs=pl.BlockSpec((BT, D), lambda i: (i, 0)),
    out_shape=jax.ShapeDtypeStruct((N, D), jnp.float32),
)


@jax.jit
def kernel(x, edge_index, W_l0, b_l0, W_r0, W_l1, b_l1, W_r1):
  pad = EPAD - E
  pad_src = jnp.arange(pad, dtype=jnp.int32) % N
  pad_dst = N + jnp.arange(pad, dtype=jnp.int32) % (NP - N)
  src = jnp.concatenate([edge_index[0], pad_src]).reshape(NW, NCH, C)
  dst = jnp.concatenate([edge_index[1], pad_dst]).reshape(NW, NCH, C)
  acc1, cnt1 = _sc_agg_counts(x, src, dst)
  cnt1 = cnt1.reshape(NC, NP)
  h1 = _tc_combine(acc1, cnt1, x, W_l0, W_r0, b_l0.reshape(1, D))
  (acc2,) = _sc_agg(h1, src, dst)
  out = _tc_combine(acc2, cnt1, h1, W_l1, W_r1, b_l1.reshape(1, D))
  return out


# speedup vs baseline: 1.2342x; 1.2342x over previous
import functools

import jax
import jax.numpy as jnp
from jax import lax
from jax.experimental import pallas as pl
from jax.experimental.pallas import tpu as pltpu
from jax.experimental.pallas import tpu_sc as plsc

N = 10000
E = 320000
D = 128

NC = 2
NS = 16
NW = NC * NS
C = 80
NCH = 126
EPAD = NW * NCH * C
NP = 10240
RPT = NP // NS
BT = 1024

_mesh = plsc.VectorSubcoreMesh(core_axis_name="c", subcore_axis_name="s")


def _make_sc(with_counts: bool):
  out_type = [jax.ShapeDtypeStruct((NC, NP, D), jnp.float32)]
  scratch = [
      pltpu.VMEM_SHARED((NP, D), jnp.float32),
      pltpu.VMEM((NCH, C), jnp.int32),
      pltpu.VMEM((NCH, C), jnp.int32),
      pltpu.VMEM((C, D), jnp.float32),
      pltpu.VMEM((C, D), jnp.float32),
      pltpu.SemaphoreType.DMA,
      pltpu.SemaphoreType.DMA,
      pltpu.SemaphoreType.DMA,
      pltpu.SemaphoreType.DMA,
  ]
  if with_counts:
    out_type.append(jax.ShapeDtypeStruct((NC * NP,), jnp.float32))
    scratch += [
        pltpu.VMEM_SHARED((NP,), jnp.float32),
        pltpu.VMEM((C,), jnp.float32),
        pltpu.VMEM((RPT,), jnp.float32),
        pltpu.SemaphoreType.DMA,
        pltpu.SemaphoreType.DMA,
    ]

  def body(table, src_i, dst_i, acc_out, cnt_out, acc_sh, src_v, dst_v,
           rows0, rows1, sem_g0, sem_g1, sem_s0, sem_s1,
           cnt_sh=None, ones_v=None, cnt_v=None, sem_c0=None, sem_c1=None):
    c = lax.axis_index("c")
    s = lax.axis_index("s")
    w = c * NS + s
    zero16 = jnp.zeros((16,), jnp.float32)

    @pl.loop(0, C)
    def _(i):
      for k in range(D // 16):
        rows0[i, pl.ds(16 * k, 16)] = zero16

    @pl.loop(0, RPT // C)
    def _(k):
      pltpu.sync_copy(rows0, acc_sh.at[pl.ds(s * RPT + k * C, C)])

    pltpu.sync_copy(src_i.at[w], src_v)
    pltpu.sync_copy(dst_i.at[w], dst_v)
    if with_counts:
      @pl.loop(0, RPT // 16)
      def _(i):
        cnt_v[pl.ds(16 * i, 16)] = zero16
      pltpu.sync_copy(cnt_v, cnt_sh.at[pl.ds(s * RPT, RPT)])
      one16 = jnp.ones((16,), jnp.float32)
      for k in range(C // 16):
        ones_v[pl.ds(16 * k, 16)] = one16
    plsc.subcore_barrier()

    pltpu.async_copy(table.at[src_v.at[0]], rows0, sem_g0)
    pltpu.async_copy(table.at[src_v.at[1]], rows1, sem_g1)

    @pl.loop(0, NCH, step=2)
    def _(j):
      pltpu.make_async_copy(table.at[src_v.at[j]], rows0, sem_g0).wait()
      pltpu.sync_copy(rows0, acc_sh.at[dst_v.at[j]], add=True)
      if with_counts:
        pltpu.sync_copy(ones_v, cnt_sh.at[dst_v.at[j]], add=True)
      pltpu.async_copy(table.at[src_v.at[(j + 2) % NCH]], rows0, sem_g0)

      j1 = j + 1
      pltpu.make_async_copy(table.at[src_v.at[j1]], rows1, sem_g1).wait()
      pltpu.sync_copy(rows1, acc_sh.at[dst_v.at[j1]], add=True)
      if with_counts:
        pltpu.sync_copy(ones_v, cnt_sh.at[dst_v.at[j1]], add=True)
      pltpu.async_copy(table.at[src_v.at[(j1 + 2) % NCH]], rows1, sem_g1)

    pltpu.make_async_copy(table.at[src_v.at[0]], rows0, sem_g0).wait()
    pltpu.make_async_copy(table.at[src_v.at[1]], rows1, sem_g1).wait()
    plsc.subcore_barrier()

    @pl.loop(0, RPT // C)
    def _(k):
      off = s * RPT + k * C
      pltpu.sync_copy(acc_sh.at[pl.ds(off, C)], rows0)
      pltpu.sync_copy(rows0, acc_out.at[c].at[pl.ds(off, C)])
    if with_counts:
      pltpu.sync_copy(cnt_sh.at[pl.ds(s * RPT, RPT)], cnt_v)
      pltpu.sync_copy(cnt_v, cnt_out.at[pl.ds(c * NP + s * RPT, RPT)])

  if with_counts:
    def body_wc(table, src_i, dst_i, acc_out, cnt_out, acc_sh, src_v,
                dst_v, rows0, rows1, sem_g0, sem_g1, sem_s0, sem_s1,
                cnt_sh, ones_v, cnt_v, sem_c0, sem_c1):
      body(table, src_i, dst_i, acc_out, cnt_out, acc_sh, src_v, dst_v,
           rows0, rows1, sem_g0, sem_g1, sem_s0, sem_s1,
           cnt_sh, ones_v, cnt_v, sem_c0, sem_c1)
    fn = body_wc
  else:
    def body_nc(table, src_i, dst_i, acc_out, acc_sh, src_v, dst_v,
                rows0, rows1, sem_g0, sem_g1, sem_s0, sem_s1):
      body(table, src_i, dst_i, acc_out, None, acc_sh, src_v, dst_v,
           rows0, rows1, sem_g0, sem_g1, sem_s0, sem_s1)
    fn = body_nc

  return pl.kernel(
      fn, out_type=out_type, mesh=_mesh, scratch_types=scratch,
      compiler_params=pltpu.CompilerParams(use_tc_tiling_on_sc=False),
      name="sc_agg_cnt" if with_counts else "sc_agg")


_sc_agg_counts = _make_sc(True)
_sc_agg = _make_sc(False)


def _tc_combine_body(acc_ref, cnt_ref, h_ref, wl_ref, wr_ref, b_ref, out_ref):
  agg = acc_ref[0] + acc_ref[1]
  cnt = jnp.sum(cnt_ref[...], axis=0)[:, None]
  mean = agg * (1.0 / jnp.maximum(cnt, 1.0))
  dn = (((1,), (1,)), ((), ()))
  out_ref[...] = (
      lax.dot_general(mean, wl_ref[...], dn, preferred_element_type=jnp.float32)
      + lax.dot_general(h_ref[...], wr_ref[...], dn,
                        preferred_element_type=jnp.float32)
      + b_ref[...])


_tc_combine = pl.pallas_call(
    _tc_combine_body,
    grid=(NP // BT,),
    in_specs=[
        pl.BlockSpec((NC, BT, D), lambda i: (0, i, 0)),
        pl.BlockSpec((NC, BT), lambda i: (0, i)),
        pl.BlockSpec((BT, D), lambda i: (i, 0)),
        pl.BlockSpec((D, D), lambda i: (0, 0)),
        pl.BlockSpec((D, D), lambda i: (0, 0)),
        pl.BlockSpec((1, D), lambda i: (0, 0)),
    ],
    out_specs=pl.BlockSpec((BT, D), lambda i: (i, 0)),
    out_shape=jax.ShapeDtypeStruct((N, D), jnp.float32),
)


@jax.jit
def kernel(x, edge_index, W_l0, b_l0, W_r0, W_l1, b_l1, W_r1):
  pad = EPAD - E
  pad_src = jnp.arange(pad, dtype=jnp.int32) % N
  pad_dst = N + jnp.arange(pad, dtype=jnp.int32) % (NP - N)
  src = jnp.concatenate([edge_index[0], pad_src]).reshape(NW, NCH, C)
  dst = jnp.concatenate([edge_index[1], pad_dst]).reshape(NW, NCH, C)
  acc1, cnt1 = _sc_agg_counts(x, src, dst)
  cnt1 = cnt1.reshape(NC, NP)
  h1 = _tc_combine(acc1, cnt1, x, W_l0, W_r0, b_l0.reshape(1, D))
  (acc2,) = _sc_agg(h1, src, dst)
  out = _tc_combine(acc2, cnt1, h1, W_l1, W_r1, b_l1.reshape(1, D))
  return out


# speedup vs baseline: 1.4289x; 1.1578x over previous
import functools

import jax
import jax.numpy as jnp
from jax import lax
from jax.experimental import pallas as pl
from jax.experimental.pallas import tpu as pltpu
from jax.experimental.pallas import tpu_sc as plsc

N = 10000
E = 320000
D = 128

NC = 2
NS = 16
NW = NC * NS
C = 128
NCH = 80
EPAD = NW * NCH * C
NP = 10240
RPT = NP // NS
BT = 1024

_mesh = plsc.VectorSubcoreMesh(core_axis_name="c", subcore_axis_name="s")


def _make_sc(with_counts: bool):
  out_type = [jax.ShapeDtypeStruct((NC, NP, D), jnp.bfloat16)]
  scratch = [
      pltpu.VMEM_SHARED((NP, D), jnp.bfloat16),
      pltpu.VMEM((NCH, C), jnp.int32),
      pltpu.VMEM((NCH, C), jnp.int32),
      pltpu.VMEM((C, D), jnp.bfloat16),
      pltpu.VMEM((C, D), jnp.bfloat16),
      pltpu.SemaphoreType.DMA,
      pltpu.SemaphoreType.DMA,
      pltpu.SemaphoreType.DMA,
      pltpu.SemaphoreType.DMA,
  ]
  if with_counts:
    out_type.append(jax.ShapeDtypeStruct((NC * NP,), jnp.float32))
    scratch += [
        pltpu.VMEM_SHARED((NP,), jnp.float32),
        pltpu.VMEM((C,), jnp.float32),
        pltpu.VMEM((RPT,), jnp.float32),
        pltpu.SemaphoreType.DMA,
        pltpu.SemaphoreType.DMA,
    ]

  def body(table, src_i, dst_i, acc_out, cnt_out, acc_sh, src_v, dst_v,
           rows0, rows1, sem_g0, sem_g1, sem_s0, sem_s1,
           cnt_sh=None, ones_v=None, cnt_v=None, sem_c0=None, sem_c1=None):
    c = lax.axis_index("c")
    s = lax.axis_index("s")
    w = c * NS + s
    zero16 = jnp.zeros((16,), jnp.float32)
    zero32 = jnp.zeros((32,), jnp.bfloat16)

    @pl.loop(0, C)
    def _(i):
      for k in range(D // 32):
        rows0[i, pl.ds(32 * k, 32)] = zero32

    @pl.loop(0, RPT // C)
    def _(k):
      pltpu.sync_copy(rows0, acc_sh.at[pl.ds(s * RPT + k * C, C)])

    pltpu.sync_copy(src_i.at[w], src_v)
    pltpu.sync_copy(dst_i.at[w], dst_v)
    if with_counts:
      @pl.loop(0, RPT // 16)
      def _(i):
        cnt_v[pl.ds(16 * i, 16)] = zero16
      pltpu.sync_copy(cnt_v, cnt_sh.at[pl.ds(s * RPT, RPT)])
      one16 = jnp.ones((16,), jnp.float32)
      for k in range(C // 16):
        ones_v[pl.ds(16 * k, 16)] = one16
    plsc.subcore_barrier()

    pltpu.async_copy(table.at[src_v.at[0]], rows0, sem_g0)
    pltpu.async_copy(table.at[src_v.at[1]], rows1, sem_g1)

    @pl.loop(0, NCH, step=2)
    def _(j):
      pltpu.make_async_copy(table.at[src_v.at[j]], rows0, sem_g0).wait()
      pltpu.sync_copy(rows0, acc_sh.at[dst_v.at[j]], add=True)
      if with_counts:
        pltpu.sync_copy(ones_v, cnt_sh.at[dst_v.at[j]], add=True)
      pltpu.async_copy(table.at[src_v.at[(j + 2) % NCH]], rows0, sem_g0)

      j1 = j + 1
      pltpu.make_async_copy(table.at[src_v.at[j1]], rows1, sem_g1).wait()
      pltpu.sync_copy(rows1, acc_sh.at[dst_v.at[j1]], add=True)
      if with_counts:
        pltpu.sync_copy(ones_v, cnt_sh.at[dst_v.at[j1]], add=True)
      pltpu.async_copy(table.at[src_v.at[(j1 + 2) % NCH]], rows1, sem_g1)

    pltpu.make_async_copy(table.at[src_v.at[0]], rows0, sem_g0).wait()
    pltpu.make_async_copy(table.at[src_v.at[1]], rows1, sem_g1).wait()
    plsc.subcore_barrier()

    @pl.loop(0, RPT // C)
    def _(k):
      off = s * RPT + k * C
      pltpu.sync_copy(acc_sh.at[pl.ds(off, C)], rows0)
      pltpu.sync_copy(rows0, acc_out.at[c].at[pl.ds(off, C)])
    if with_counts:
      pltpu.sync_copy(cnt_sh.at[pl.ds(s * RPT, RPT)], cnt_v)
      pltpu.sync_copy(cnt_v, cnt_out.at[pl.ds(c * NP + s * RPT, RPT)])

  if with_counts:
    def body_wc(table, src_i, dst_i, acc_out, cnt_out, acc_sh, src_v,
                dst_v, rows0, rows1, sem_g0, sem_g1, sem_s0, sem_s1,
                cnt_sh, ones_v, cnt_v, sem_c0, sem_c1):
      body(table, src_i, dst_i, acc_out, cnt_out, acc_sh, src_v, dst_v,
           rows0, rows1, sem_g0, sem_g1, sem_s0, sem_s1,
           cnt_sh, ones_v, cnt_v, sem_c0, sem_c1)
    fn = body_wc
  else:
    def body_nc(table, src_i, dst_i, acc_out, acc_sh, src_v, dst_v,
                rows0, rows1, sem_g0, sem_g1, sem_s0, sem_s1):
      body(table, src_i, dst_i, acc_out, None, acc_sh, src_v, dst_v,
           rows0, rows1, sem_g0, sem_g1, sem_s0, sem_s1)
    fn = body_nc

  return pl.kernel(
      fn, out_type=out_type, mesh=_mesh, scratch_types=scratch,
      compiler_params=pltpu.CompilerParams(use_tc_tiling_on_sc=False),
      name="sc_agg_cnt" if with_counts else "sc_agg")


_sc_agg_counts = _make_sc(True)
_sc_agg = _make_sc(False)


def _tc_combine_body(acc_ref, cnt_ref, h_ref, wl_ref, wr_ref, b_ref, out_ref):
  agg = (acc_ref[0].astype(jnp.float32) + acc_ref[1].astype(jnp.float32))
  cnt = jnp.sum(cnt_ref[...], axis=0)[:, None]
  mean = agg * (1.0 / jnp.maximum(cnt, 1.0))
  dn = (((1,), (1,)), ((), ()))
  out_ref[...] = (
      lax.dot_general(mean, wl_ref[...], dn, preferred_element_type=jnp.float32)
      + lax.dot_general(h_ref[...], wr_ref[...], dn,
                        preferred_element_type=jnp.float32)
      + b_ref[...])


_tc_combine = pl.pallas_call(
    _tc_combine_body,
    grid=(NP // BT,),
    in_specs=[
        pl.BlockSpec((NC, BT, D), lambda i: (0, i, 0)),
        pl.BlockSpec((NC, BT), lambda i: (0, i)),
        pl.BlockSpec((BT, D), lambda i: (i, 0)),
        pl.BlockSpec((D, D), lambda i: (0, 0)),
        pl.BlockSpec((D, D), lambda i: (0, 0)),
        pl.BlockSpec((1, D), lambda i: (0, 0)),
    ],
    out_specs=pl.BlockSpec((BT, D), lambda i: (i, 0)),
    out_shape=jax.ShapeDtypeStruct((N, D), jnp.float32),
)


@jax.jit
def kernel(x, edge_index, W_l0, b_l0, W_r0, W_l1, b_l1, W_r1):
  pad = EPAD - E
  pad_src = jnp.arange(pad, dtype=jnp.int32) % N
  pad_dst = N + jnp.arange(pad, dtype=jnp.int32) % (NP - N)
  src = jnp.concatenate([edge_index[0], pad_src]).reshape(NW, NCH, C)
  dst = jnp.concatenate([edge_index[1], pad_dst]).reshape(NW, NCH, C)
  acc1, cnt1 = _sc_agg_counts(x.astype(jnp.bfloat16), src, dst)
  cnt1 = cnt1.reshape(NC, NP)
  h1 = _tc_combine(acc1, cnt1, x, W_l0, W_r0, b_l0.reshape(1, D))
  (acc2,) = _sc_agg(h1.astype(jnp.bfloat16), src, dst)
  out = _tc_combine(acc2, cnt1, h1, W_l1, W_r1, b_l1.reshape(1, D))
  return out
